# baseline (speedup 1.0000x reference)
import functools

import jax
import jax.numpy as jnp
from jax import lax
from jax.experimental import pallas as pl
from jax.experimental.pallas import tpu as pltpu
from jax.experimental.pallas import tpu_sc as plsc

N = 2_000_000
B = 4
F = 8
R = 64
NW = 32
L = 16
C = 4096
LEN_A = 62504
LEN_B = 62496
N_CHUNKS = 16


def _sc_body(pts_hbm, bidx_hbm, ts_hbm, table_hbm, kf_hbm, out_hbm,
             pts_v, bidx_v, ts_v, idx_v, gath_v, kf_v, sem):
    wid = lax.axis_index("s") * 2 + lax.axis_index("c")
    base = jnp.where(wid < 16, wid * LEN_A, 16 * LEN_A + (wid - 16) * LEN_B)
    lenw = jnp.where(wid < 16, LEN_A, LEN_B)

    pltpu.sync_copy(kf_hbm, kf_v)
    lane = lax.iota(jnp.int32, L)
    lane3 = lane * 3
    kfb = [kf_v[pl.ds(k * L, L)] for k in range(F)]

    def compute_chunk():
        def body(j, carry):
            off = j * L
            bv = bidx_v[pl.ds(off, L)]
            tv = ts_v[pl.ds(off, L)]
            p = off * 3 + lane3
            px = plsc.load_gather(pts_v, [p])
            py = plsc.load_gather(pts_v, [p + 1])
            pz = plsc.load_gather(pts_v, [p + 2])
            gx = jnp.clip(((px / 2.0 + 0.5) * R).astype(jnp.int32), 0, R - 1)
            gy = jnp.clip(((py / 2.0 + 0.5) * R).astype(jnp.int32), 0, R - 1)
            gz = jnp.clip(((pz / 2.0 + 0.5) * R).astype(jnp.int32), 0, R - 1)
            fidx = jnp.zeros((L,), jnp.int32)
            for k in range(1, F):
                fidx = fidx + ((tv - kfb[k - 1]) > (kfb[k] - tv)).astype(
                    jnp.int32)
            flat = ((bv * F + fidx) * R + gx) * (R * R) + gy * R + gz
            idx_v[pl.ds(off, L)] = flat
            return carry
        lax.fori_loop(0, C // L, body, 0, unroll=2)

    for i in range(N_CHUNKS):
        start = jnp.minimum(i * C, lenw - C)
        g = base + start
        pltpu.sync_copy(pts_hbm.at[pl.ds(g * 3, 3 * C)], pts_v)
        pltpu.sync_copy(bidx_hbm.at[pl.ds(g, C)], bidx_v)
        pltpu.sync_copy(ts_hbm.at[pl.ds(g, C)], ts_v)
        compute_chunk()
        pltpu.async_copy(table_hbm.at[idx_v], gath_v, sem).wait()
        pltpu.sync_copy(gath_v, out_hbm.at[pl.ds(g, C)])


_sc_kernel = functools.partial(
    pl.kernel,
    out_type=jax.ShapeDtypeStruct((N,), jnp.float32),
    mesh=plsc.VectorSubcoreMesh(core_axis_name="c", subcore_axis_name="s"),
    compiler_params=pltpu.CompilerParams(needs_layout_passes=False),
    scratch_types=[
        pltpu.VMEM((3 * C,), jnp.float32),
        pltpu.VMEM((C,), jnp.int32),
        pltpu.VMEM((C,), jnp.float32),
        pltpu.VMEM((C,), jnp.int32),
        pltpu.VMEM((C,), jnp.float32),
        pltpu.VMEM((F * L,), jnp.float32),
        pltpu.SemaphoreType.DMA,
    ],
)(_sc_body)


def kernel(pts, bidx, ts, tmp_flat_occ_grid, ts_keyframes):
    table = tmp_flat_occ_grid.reshape(-1)
    ptsf = pts.reshape(-1)
    kfb_host = jnp.repeat(ts_keyframes, L)
    return _sc_kernel(ptsf, bidx, ts, table, kfb_host)


# speedup vs baseline: 1.0320x; 1.0320x over previous
import functools

import jax
import jax.numpy as jnp
from jax import lax
from jax.experimental import pallas as pl
from jax.experimental.pallas import tpu as pltpu
from jax.experimental.pallas import tpu_sc as plsc

N = 2_000_000
B = 4
F = 8
R = 64


ROWS = N // 128
RB = 512
GRID = -(-ROWS // RB)


def _idx_body(pts3_ref, bidx_ref, ts_ref, kfb_ref, idx_ref):
    p = pts3_ref[...]
    q = jnp.clip(((p / 2.0 + 0.5) * R).astype(jnp.int32), 0, R - 1)
    i3 = lax.broadcasted_iota(jnp.int32, (3 * 128, 128), 0)
    j3 = lax.broadcasted_iota(jnp.int32, (3 * 128, 128), 1)
    wv = jnp.where(i3 % 3 == 0, float(R * R),
                   jnp.where(i3 % 3 == 1, float(R), 1.0))
    w = jnp.where(i3 // 3 == j3, wv, 0.0)
    voxel = jax.lax.dot_general(
        q.astype(jnp.float32), w, (((1,), (0,)), ((), ())),
        preferred_element_type=jnp.float32).astype(jnp.int32)
    tv = ts_ref[...]
    fidx = jnp.zeros(tv.shape, jnp.int32)
    for k in range(1, F):
        left = kfb_ref[k - 1, :][None, :]
        right = kfb_ref[k, :][None, :]
        fidx = fidx + ((tv - left) > (right - tv)).astype(jnp.int32)
    bv = bidx_ref[...]
    idx_ref[...] = (bv * F + fidx) * (R * R * R) + voxel


_idx_kernel = pl.pallas_call(
    _idx_body,
    grid=(GRID,),
    in_specs=[
        pl.BlockSpec((RB, 3 * 128), lambda i: (i, 0)),
        pl.BlockSpec((RB, 128), lambda i: (i, 0)),
        pl.BlockSpec((RB, 128), lambda i: (i, 0)),
        pl.BlockSpec((F, 128), lambda i: (0, 0)),
    ],
    out_specs=pl.BlockSpec((RB, 128), lambda i: (i, 0)),
    out_shape=jax.ShapeDtypeStruct((ROWS, 128), jnp.int32),
)


NW = 32
L = 16
C = 16384
LEN_A = 62504
LEN_B = 62496
N_CHUNKS = 4


def _gather_body(idx_hbm, table_hbm, out_hbm,
                 idx_v, gath_v0, gath_v1, s_in, s_g, s_o0, s_o1):
    wid = lax.axis_index("s") * 2 + lax.axis_index("c")
    base = jnp.where(wid < 16, wid * LEN_A, 16 * LEN_A + (wid - 16) * LEN_B)
    lenw = jnp.where(wid < 16, LEN_A, LEN_B)
    loc = [jnp.minimum(i * C, lenw - C) for i in range(N_CHUNKS)]

    h_a = pltpu.async_copy(idx_hbm.at[pl.ds(base, LEN_B)],
                           idx_v.at[pl.ds(0, LEN_B)], s_in)
    tail = jnp.where(wid < 16, base + LEN_B, 0)
    h_b = pltpu.async_copy(idx_hbm.at[pl.ds(tail, 8)],
                           idx_v.at[pl.ds(LEN_B, 8)], s_in)
    h_a.wait()
    h_b.wait()

    gath_v = [gath_v0, gath_v1]
    s_out = [s_o0, s_o1]
    h_out = [None, None]
    for i in range(N_CHUNKS):
        b = i % 2
        if i >= 2:
            h_out[b].wait()
        pltpu.async_copy(table_hbm.at[idx_v.at[pl.ds(loc[i], C)]],
                         gath_v[b], s_g).wait()
        h_out[b] = pltpu.async_copy(
            gath_v[b], out_hbm.at[pl.ds(base + loc[i], C)], s_out[b])
    h_out[0].wait()
    h_out[1].wait()


_gather_kernel = functools.partial(
    pl.kernel,
    out_type=jax.ShapeDtypeStruct((N,), jnp.float32),
    mesh=plsc.VectorSubcoreMesh(core_axis_name="c", subcore_axis_name="s"),
    compiler_params=pltpu.CompilerParams(needs_layout_passes=False),
    scratch_types=[
        pltpu.VMEM((LEN_A,), jnp.int32),
        pltpu.VMEM((C,), jnp.float32),
        pltpu.VMEM((C,), jnp.float32),
        pltpu.SemaphoreType.DMA,
        pltpu.SemaphoreType.DMA,
        pltpu.SemaphoreType.DMA,
        pltpu.SemaphoreType.DMA,
    ],
)(_gather_body)


def kernel(pts, bidx, ts, tmp_flat_occ_grid, ts_keyframes):
    pts3 = pts.reshape(ROWS, 3 * 128)
    bidx2 = bidx.reshape(ROWS, 128)
    ts2 = ts.reshape(ROWS, 128)
    kfb = jnp.broadcast_to(ts_keyframes[:, None], (F, 128))
    idx = _idx_kernel(pts3, bidx2, ts2, kfb).reshape(N)
    table = tmp_flat_occ_grid.reshape(-1)
    return _gather_kernel(idx, table)


# speedup vs baseline: 22.5786x; 21.8791x over previous
import functools

import jax
import jax.numpy as jnp
from jax import lax
from jax.experimental import pallas as pl
from jax.experimental.pallas import tpu as pltpu
from jax.experimental.pallas import tpu_sc as plsc

N = 2_000_000
B = 4
F = 8
R = 64


ROWS = N // 128
RB = 512
GRID = -(-ROWS // RB)


def _idx_body(x_ref, y_ref, z_ref, bidx_ref, ts_ref, kfb_ref, idx_ref):
    gx = jnp.clip(((x_ref[...] / 2.0 + 0.5) * R).astype(jnp.int32), 0, R - 1)
    gy = jnp.clip(((y_ref[...] / 2.0 + 0.5) * R).astype(jnp.int32), 0, R - 1)
    gz = jnp.clip(((z_ref[...] / 2.0 + 0.5) * R).astype(jnp.int32), 0, R - 1)
    tv = ts_ref[...]
    fidx = jnp.zeros(tv.shape, jnp.int32)
    for k in range(1, F):
        left = kfb_ref[k - 1, :][None, :]
        right = kfb_ref[k, :][None, :]
        fidx = fidx + ((tv - left) > (right - tv)).astype(jnp.int32)
    bv = bidx_ref[...]
    idx_ref[...] = (((bv * F + fidx) * R + gx) * R + gy) * 128 + gz


_idx_kernel = pl.pallas_call(
    _idx_body,
    grid=(GRID,),
    in_specs=[
        pl.BlockSpec((RB, 128), lambda i: (i, 0)),
        pl.BlockSpec((RB, 128), lambda i: (i, 0)),
        pl.BlockSpec((RB, 128), lambda i: (i, 0)),
        pl.BlockSpec((RB, 128), lambda i: (i, 0)),
        pl.BlockSpec((RB, 128), lambda i: (i, 0)),
        pl.BlockSpec((F, 128), lambda i: (0, 0)),
    ],
    out_specs=pl.BlockSpec((RB, 128), lambda i: (i, 0)),
    out_shape=jax.ShapeDtypeStruct((ROWS, 128), jnp.int32),
)


TROWS = B * F * R * R
TRB = 2048
TGRID = TROWS // TRB


def _pad_body(tab_ref, out_ref):
    t = tab_ref[...]
    out_ref[...] = jnp.concatenate([t, jnp.zeros_like(t)], axis=1)


_pad_kernel = pl.pallas_call(
    _pad_body,
    grid=(TGRID,),
    in_specs=[pl.BlockSpec((TRB, R), lambda i: (i, 0))],
    out_specs=pl.BlockSpec((TRB, 128), lambda i: (i, 0)),
    out_shape=jax.ShapeDtypeStruct((TROWS, 128), jnp.float32),
)


NW = 32
L = 16
C = 16384
LEN_A = 62504
LEN_B = 62496
N_CHUNKS = 4


def _gather_body(idx_hbm, table_hbm, out_hbm,
                 idx_v, gath_v0, gath_v1, s_in, s_g, s_o0, s_o1):
    wid = lax.axis_index("s") * 2 + lax.axis_index("c")
    base = jnp.where(wid < 16, wid * LEN_A, 16 * LEN_A + (wid - 16) * LEN_B)
    lenw = jnp.where(wid < 16, LEN_A, LEN_B)
    loc = [jnp.minimum(i * C, lenw - C) for i in range(N_CHUNKS)]

    h_a = pltpu.async_copy(idx_hbm.at[pl.ds(base, LEN_B)],
                           idx_v.at[pl.ds(0, LEN_B)], s_in)
    tail = jnp.where(wid < 16, base + LEN_B, 0)
    h_b = pltpu.async_copy(idx_hbm.at[pl.ds(tail, 8)],
                           idx_v.at[pl.ds(LEN_B, 8)], s_in)
    h_a.wait()
    h_b.wait()

    gath_v = [gath_v0, gath_v1]
    s_out = [s_o0, s_o1]
    h_out = [None, None]
    for i in range(N_CHUNKS):
        b = i % 2
        if i >= 2:
            h_out[b].wait()
        pltpu.async_copy(table_hbm.at[idx_v.at[pl.ds(loc[i], C)]],
                         gath_v[b], s_g).wait()
        h_out[b] = pltpu.async_copy(
            gath_v[b], out_hbm.at[pl.ds(base + loc[i], C)], s_out[b])
    h_out[0].wait()
    h_out[1].wait()


_gather_kernel = functools.partial(
    pl.kernel,
    out_type=jax.ShapeDtypeStruct((N,), jnp.float32),
    mesh=plsc.VectorSubcoreMesh(core_axis_name="c", subcore_axis_name="s"),
    compiler_params=pltpu.CompilerParams(needs_layout_passes=False),
    scratch_types=[
        pltpu.VMEM((LEN_A,), jnp.int32),
        pltpu.VMEM((C,), jnp.float32),
        pltpu.VMEM((C,), jnp.float32),
        pltpu.SemaphoreType.DMA,
        pltpu.SemaphoreType.DMA,
        pltpu.SemaphoreType.DMA,
        pltpu.SemaphoreType.DMA,
    ],
)(_gather_body)


def kernel(pts, bidx, ts, tmp_flat_occ_grid, ts_keyframes):
    x2 = pts[:, 0].reshape(ROWS, 128)
    y2 = pts[:, 1].reshape(ROWS, 128)
    z2 = pts[:, 2].reshape(ROWS, 128)
    bidx2 = bidx.reshape(ROWS, 128)
    ts2 = ts.reshape(ROWS, 128)
    kfb = jnp.broadcast_to(ts_keyframes[:, None], (F, 128))
    idx = _idx_kernel(x2, y2, z2, bidx2, ts2, kfb).reshape(N)
    table2 = tmp_flat_occ_grid.reshape(TROWS, R)
    table = _pad_kernel(table2).reshape(TROWS * 128)
    return _gather_kernel(idx, table)


# speedup vs baseline: 24.8361x; 1.1000x over previous
import functools

import jax
import jax.numpy as jnp
from jax import lax
from jax.experimental import pallas as pl
from jax.experimental.pallas import tpu as pltpu
from jax.experimental.pallas import tpu_sc as plsc

N = 2_000_000
B = 4
F = 8
R = 64


ROWS = N // 128
RB = 512
GRID = -(-ROWS // RB)


def _idx_body(ptsT_ref, bidx_ref, ts_ref, kfb_ref, idx_ref):
    gx = jnp.clip(((ptsT_ref[0] / 2.0 + 0.5) * R).astype(jnp.int32), 0, R - 1)
    gy = jnp.clip(((ptsT_ref[1] / 2.0 + 0.5) * R).astype(jnp.int32), 0, R - 1)
    gz = jnp.clip(((ptsT_ref[2] / 2.0 + 0.5) * R).astype(jnp.int32), 0, R - 1)
    tv = ts_ref[...]
    fidx = jnp.zeros(tv.shape, jnp.int32)
    for k in range(1, F):
        left = kfb_ref[k - 1, :][None, :]
        right = kfb_ref[k, :][None, :]
        fidx = fidx + ((tv - left) > (right - tv)).astype(jnp.int32)
    bv = bidx_ref[...]
    idx_ref[...] = (((bv * F + fidx) * R + gx) * R + gy) * 128 + gz


_idx_kernel = pl.pallas_call(
    _idx_body,
    grid=(GRID,),
    in_specs=[
        pl.BlockSpec((3, RB, 128), lambda i: (0, i, 0)),
        pl.BlockSpec((RB, 128), lambda i: (i, 0)),
        pl.BlockSpec((RB, 128), lambda i: (i, 0)),
        pl.BlockSpec((F, 128), lambda i: (0, 0)),
    ],
    out_specs=pl.BlockSpec((RB, 128), lambda i: (i, 0)),
    out_shape=jax.ShapeDtypeStruct((ROWS, 128), jnp.int32),
)


TROWS = B * F * R * R
TRB = 2048
TGRID = TROWS // TRB


def _pad_body(tab_ref, out_ref):
    t = tab_ref[...]
    out_ref[...] = jnp.concatenate([t, t], axis=1)


_pad_kernel = pl.pallas_call(
    _pad_body,
    grid=(TGRID,),
    in_specs=[pl.BlockSpec((TRB, R), lambda i: (i, 0))],
    out_specs=pl.BlockSpec((TRB, 128), lambda i: (i, 0)),
    out_shape=jax.ShapeDtypeStruct((TROWS, 128), jnp.float32),
)


NW = 32
L = 16
LEN_SHORT = 62496
LEN_LONG = 62504
C = 31256
N_CHUNKS = 2


def _gather_body(idx_hbm, table_hbm, out_hbm,
                 idx_v, gath_v0, gath_v1, s_in, s_g, s_o0, s_o1):
    wid = lax.axis_index("s") * 2 + lax.axis_index("c")
    base = jnp.where(wid < 16, wid * LEN_SHORT,
                     16 * LEN_SHORT + (wid - 16) * LEN_LONG)
    pltpu.async_copy(idx_hbm.at[pl.ds(base, LEN_LONG)], idx_v, s_in).wait()

    gath_v = [gath_v0, gath_v1]
    s_out = [s_o0, s_o1]
    loc = [0, LEN_LONG - C]
    h_out = [None, None]
    for i in range(N_CHUNKS):
        pltpu.async_copy(table_hbm.at[idx_v.at[pl.ds(loc[i], C)]],
                         gath_v[i], s_g).wait()
        h_out[i] = pltpu.async_copy(
            gath_v[i], out_hbm.at[pl.ds(base + loc[i], C)], s_out[i])
    h_out[0].wait()
    h_out[1].wait()


_gather_kernel = functools.partial(
    pl.kernel,
    out_type=jax.ShapeDtypeStruct((N,), jnp.float32),
    mesh=plsc.VectorSubcoreMesh(core_axis_name="c", subcore_axis_name="s"),
    compiler_params=pltpu.CompilerParams(needs_layout_passes=False),
    scratch_types=[
        pltpu.VMEM((LEN_LONG,), jnp.int32),
        pltpu.VMEM((C,), jnp.float32),
        pltpu.VMEM((C,), jnp.float32),
        pltpu.SemaphoreType.DMA,
        pltpu.SemaphoreType.DMA,
        pltpu.SemaphoreType.DMA,
        pltpu.SemaphoreType.DMA,
    ],
)(_gather_body)


def kernel(pts, bidx, ts, tmp_flat_occ_grid, ts_keyframes):
    ptsT = jnp.transpose(pts).reshape(3, ROWS, 128)
    bidx2 = bidx.reshape(ROWS, 128)
    ts2 = ts.reshape(ROWS, 128)
    kfb = jnp.broadcast_to(ts_keyframes[:, None], (F, 128))
    idx = _idx_kernel(ptsT, bidx2, ts2, kfb).reshape(N)
    table2 = tmp_flat_occ_grid.reshape(TROWS, R)
    table = _pad_kernel(table2).reshape(TROWS * 128)
    return _gather_kernel(idx, table)


# speedup vs baseline: 25.7802x; 1.0380x over previous
import functools

import jax
import jax.numpy as jnp
from jax import lax
from jax.experimental import pallas as pl
from jax.experimental.pallas import tpu as pltpu
from jax.experimental.pallas import tpu_sc as plsc

N = 2_000_000
B = 4
F = 8
R = 64
NW = 32

P1 = 1_000_192
P2 = N - P1


RB = 2048


def _idx_body(ptsT_ref, bidx_ref, ts_ref, kfb_ref, idx_ref):
    gx = jnp.clip(((ptsT_ref[0] / 2.0 + 0.5) * R).astype(jnp.int32), 0, R - 1)
    gy = jnp.clip(((ptsT_ref[1] / 2.0 + 0.5) * R).astype(jnp.int32), 0, R - 1)
    gz = jnp.clip(((ptsT_ref[2] / 2.0 + 0.5) * R).astype(jnp.int32), 0, R - 1)
    tv = ts_ref[...]
    fidx = jnp.zeros(tv.shape, jnp.int32)
    for k in range(1, F):
        left = kfb_ref[k - 1, :][None, :]
        right = kfb_ref[k, :][None, :]
        fidx = fidx + ((tv - left) > (right - tv)).astype(jnp.int32)
    bv = bidx_ref[...]
    idx_ref[...] = (((bv * F + fidx) * R + gx) * R + gy) * 128 + gz


def _make_idx_kernel(rows):
    grid = -(-rows // RB)
    return pl.pallas_call(
        _idx_body,
        grid=(grid,),
        in_specs=[
            pl.BlockSpec((3, RB, 128), lambda i: (0, i, 0)),
            pl.BlockSpec((RB, 128), lambda i: (i, 0)),
            pl.BlockSpec((RB, 128), lambda i: (i, 0)),
            pl.BlockSpec((F, 128), lambda i: (0, 0)),
        ],
        out_specs=pl.BlockSpec((RB, 128), lambda i: (i, 0)),
        out_shape=jax.ShapeDtypeStruct((rows, 128), jnp.int32),
    )


_idx_kernel_1 = _make_idx_kernel(P1 // 128)
_idx_kernel_2 = _make_idx_kernel(P2 // 128)


TROWS = B * F * R * R
TRB = 16384
TGRID = TROWS // TRB


def _pad_body(tab_ref, out_ref):
    t = tab_ref[...]
    out_ref[...] = jnp.concatenate([t, t], axis=1)


_pad_kernel = pl.pallas_call(
    _pad_body,
    grid=(TGRID,),
    in_specs=[pl.BlockSpec((TRB, R), lambda i: (i, 0))],
    out_specs=pl.BlockSpec((TRB, 128), lambda i: (i, 0)),
    out_shape=jax.ShapeDtypeStruct((TROWS, 128), jnp.float32),
)


def _make_gather_kernel(total, len_short, len_long):
    assert 16 * len_short + 16 * len_long == total
    assert len_short % 8 == 0 and len_long % 8 == 0
    c = -(-(len_long // 2) // 8) * 8
    loc = [0, len_long - c]

    def body(idx_hbm, table_hbm, out_hbm,
             idx_v, gath_v0, gath_v1, s_in, s_g, s_o0, s_o1):
        wid = lax.axis_index("s") * 2 + lax.axis_index("c")
        base = jnp.where(wid < 16, wid * len_short,
                         16 * len_short + (wid - 16) * len_long)
        pltpu.async_copy(idx_hbm.at[pl.ds(base, len_long)], idx_v,
                         s_in).wait()
        gath_v = [gath_v0, gath_v1]
        s_out = [s_o0, s_o1]
        h_out = [None, None]
        for i in range(2):
            pltpu.async_copy(table_hbm.at[idx_v.at[pl.ds(loc[i], c)]],
                             gath_v[i], s_g).wait()
            h_out[i] = pltpu.async_copy(
                gath_v[i], out_hbm.at[pl.ds(base + loc[i], c)], s_out[i])
        h_out[0].wait()
        h_out[1].wait()

    return functools.partial(
        pl.kernel,
        out_type=jax.ShapeDtypeStruct((total,), jnp.float32),
        mesh=plsc.VectorSubcoreMesh(core_axis_name="c", subcore_axis_name="s"),
        compiler_params=pltpu.CompilerParams(needs_layout_passes=False),
        scratch_types=[
            pltpu.VMEM((len_long,), jnp.int32),
            pltpu.VMEM((c,), jnp.float32),
            pltpu.VMEM((c,), jnp.float32),
            pltpu.SemaphoreType.DMA,
            pltpu.SemaphoreType.DMA,
            pltpu.SemaphoreType.DMA,
            pltpu.SemaphoreType.DMA,
        ],
    )(body)


_gather_kernel_1 = _make_gather_kernel(P1, 31256, 31256)
_gather_kernel_2 = _make_gather_kernel(P2, 31240, 31248)


def _half(pts, bidx, ts, kfb, lo, hi, idx_kernel):
    rows = (hi - lo) // 128
    ptsT = jnp.transpose(pts[lo:hi]).reshape(3, rows, 128)
    bidx2 = bidx[lo:hi].reshape(rows, 128)
    ts2 = ts[lo:hi].reshape(rows, 128)
    return idx_kernel(ptsT, bidx2, ts2, kfb).reshape(hi - lo)


def kernel(pts, bidx, ts, tmp_flat_occ_grid, ts_keyframes):
    kfb = jnp.broadcast_to(ts_keyframes[:, None], (F, 128))
    table2 = tmp_flat_occ_grid.reshape(TROWS, R)
    table = _pad_kernel(table2).reshape(TROWS * 128)
    idx1 = _half(pts, bidx, ts, kfb, 0, P1, _idx_kernel_1)
    out1 = _gather_kernel_1(idx1, table)
    idx2 = _half(pts, bidx, ts, kfb, P1, N, _idx_kernel_2)
    out2 = _gather_kernel_2(idx2, table)
    return jnp.concatenate([out1, out2])


# speedup vs baseline: 29.1109x; 1.1292x over previous
import functools

import jax
import jax.numpy as jnp
from jax import lax
from jax.experimental import pallas as pl
from jax.experimental.pallas import tpu as pltpu
from jax.experimental.pallas import tpu_sc as plsc

N = 2_000_000
B = 4
F = 8
R = 64


ROWS = N // 128
RB = 4096
GRID = -(-ROWS // RB)


def _idx_body(ptsT_ref, bidx_ref, ts_ref, kfb_ref, idx_ref):
    gx = jnp.clip(((ptsT_ref[0] / 2.0 + 0.5) * R).astype(jnp.int32), 0, R - 1)
    gy = jnp.clip(((ptsT_ref[1] / 2.0 + 0.5) * R).astype(jnp.int32), 0, R - 1)
    gz = jnp.clip(((ptsT_ref[2] / 2.0 + 0.5) * R).astype(jnp.int32), 0, R - 1)
    tv = ts_ref[...]
    fidx = jnp.zeros(tv.shape, jnp.int32)
    for k in range(1, F):
        left = kfb_ref[k - 1, :][None, :]
        right = kfb_ref[k, :][None, :]
        fidx = fidx + ((tv - left) > (right - tv)).astype(jnp.int32)
    bv = bidx_ref[...]
    idx_ref[...] = (((bv * F + fidx) * R + gx) * R + gy) * 128 + gz


_idx_kernel = pl.pallas_call(
    _idx_body,
    grid=(GRID,),
    in_specs=[
        pl.BlockSpec((3, RB, 128), lambda i: (0, i, 0)),
        pl.BlockSpec((RB, 128), lambda i: (i, 0)),
        pl.BlockSpec((RB, 128), lambda i: (i, 0)),
        pl.BlockSpec((F, 128), lambda i: (0, 0)),
    ],
    out_specs=pl.BlockSpec((RB, 128), lambda i: (i, 0)),
    out_shape=jax.ShapeDtypeStruct((ROWS, 128), jnp.int32),
)


TROWS = B * F * R * R
TRB = 16384
TGRID = TROWS // TRB


def _pad_body(tab_ref, out_ref):
    t = tab_ref[...]
    out_ref[...] = jnp.concatenate([t, t], axis=1)


_pad_kernel = pl.pallas_call(
    _pad_body,
    grid=(TGRID,),
    in_specs=[pl.BlockSpec((TRB, R), lambda i: (i, 0))],
    out_specs=pl.BlockSpec((TRB, 128), lambda i: (i, 0)),
    out_shape=jax.ShapeDtypeStruct((TROWS, 128), jnp.float32),
)


NW = 32
L = 16
LEN_SHORT = 62496
LEN_LONG = 62504
C = 31256
N_CHUNKS = 2


def _gather_body(idx_hbm, table_hbm, out_hbm,
                 idx_v, gath_v0, gath_v1, s_in, s_g, s_o0, s_o1):
    wid = lax.axis_index("s") * 2 + lax.axis_index("c")
    base = jnp.where(wid < 16, wid * LEN_SHORT,
                     16 * LEN_SHORT + (wid - 16) * LEN_LONG)
    pltpu.async_copy(idx_hbm.at[pl.ds(base, LEN_LONG)], idx_v, s_in).wait()

    gath_v = [gath_v0, gath_v1]
    s_out = [s_o0, s_o1]
    loc = [0, LEN_LONG - C]
    h_out = [None, None]
    for i in range(N_CHUNKS):
        pltpu.async_copy(table_hbm.at[idx_v.at[pl.ds(loc[i], C)]],
                         gath_v[i], s_g).wait()
        h_out[i] = pltpu.async_copy(
            gath_v[i], out_hbm.at[pl.ds(base + loc[i], C)], s_out[i])
    h_out[0].wait()
    h_out[1].wait()


_gather_kernel = functools.partial(
    pl.kernel,
    out_type=jax.ShapeDtypeStruct((N,), jnp.float32),
    mesh=plsc.VectorSubcoreMesh(core_axis_name="c", subcore_axis_name="s"),
    compiler_params=pltpu.CompilerParams(needs_layout_passes=False),
    scratch_types=[
        pltpu.VMEM((LEN_LONG,), jnp.int32),
        pltpu.VMEM((C,), jnp.float32),
        pltpu.VMEM((C,), jnp.float32),
        pltpu.SemaphoreType.DMA,
        pltpu.SemaphoreType.DMA,
        pltpu.SemaphoreType.DMA,
        pltpu.SemaphoreType.DMA,
    ],
)(_gather_body)


def kernel(pts, bidx, ts, tmp_flat_occ_grid, ts_keyframes):
    ptsT = jnp.transpose(pts).reshape(3, ROWS, 128)
    bidx2 = bidx.reshape(ROWS, 128)
    ts2 = ts.reshape(ROWS, 128)
    kfb = jnp.broadcast_to(ts_keyframes[:, None], (F, 128))
    idx = _idx_kernel(ptsT, bidx2, ts2, kfb).reshape(N)
    table2 = tmp_flat_occ_grid.reshape(TROWS, R)
    table = _pad_kernel(table2).reshape(TROWS * 128)
    return _gather_kernel(idx, table)
